# baseline (speedup 1.0000x reference)
import functools

import jax
import jax.numpy as jnp
from jax import lax
from jax.experimental import pallas as pl
from jax.experimental.pallas import tpu as pltpu
from jax.experimental.pallas import tpu_sc as plsc

_N = 10000
_E = 320000
_D = 512

_NTILE = 32
_NTASK = 64
_R = 160
_NPAD = _NTASK * _R
_CHP = 12800
_NCHP = _E // _CHP
_LSZP = _CHP // 2 + 16
_T2 = _NTASK * 2
_BK = 1024
_ECAP = 160768
_NEG = -3.0e38


def _matmul(a, w, out_dtype=jnp.float32):
    m, k = a.shape
    do = w.shape[1]
    bm = 400

    def body(a_ref, w_ref, o_ref):
        o_ref[...] = jnp.dot(a_ref[...], w_ref[...],
                             preferred_element_type=jnp.float32,
                             precision=jax.lax.Precision.HIGHEST
                             ).astype(out_dtype)

    return pl.pallas_call(
        body,
        grid=(m // bm,),
        in_specs=[
            pl.BlockSpec((bm, k), lambda i: (i, 0)),
            pl.BlockSpec((k, do), lambda i: (0, 0)),
        ],
        out_specs=pl.BlockSpec((bm, do), lambda i: (i, 0)),
        out_shape=jax.ShapeDtypeStruct((m, do), out_dtype),
    )(a, w)


def _concat_matmul(a, b, wt, wb, relu):
    m, ka = a.shape
    kb = b.shape[1]
    do = wt.shape[1]
    bm = 400

    def body(a_ref, b_ref, wt_ref, wb_ref, o_ref):
        acc = jnp.dot(a_ref[...], wt_ref[...],
                      preferred_element_type=jnp.float32,
                      precision=jax.lax.Precision.HIGHEST)
        acc = acc + jnp.dot(b_ref[...].astype(jnp.float32), wb_ref[...],
                            preferred_element_type=jnp.float32,
                            precision=jax.lax.Precision.HIGHEST)
        if relu:
            acc = jnp.maximum(acc, 0.0)
        o_ref[...] = acc

    return pl.pallas_call(
        body,
        grid=(m // bm,),
        in_specs=[
            pl.BlockSpec((bm, ka), lambda i: (i, 0)),
            pl.BlockSpec((bm, kb), lambda i: (i, 0)),
            pl.BlockSpec((ka, do), lambda i: (0, 0)),
            pl.BlockSpec((kb, do), lambda i: (0, 0)),
        ],
        out_specs=pl.BlockSpec((bm, do), lambda i: (i, 0)),
        out_shape=jax.ShapeDtypeStruct((m, do), jnp.float32),
    )(a, b, wt, wb)


def _prep_sc(src, dst):
    mesh = plsc.VectorSubcoreMesh(core_axis_name="c", subcore_axis_name="s")

    @functools.partial(
        pl.kernel,
        mesh=mesh,
        out_type=(
            jax.ShapeDtypeStruct((_T2 * _ECAP,), jnp.int32),
            jax.ShapeDtypeStruct((_T2 * _ECAP,), jnp.int32),
            jax.ShapeDtypeStruct((_T2 * 16,), jnp.int32),
        ),
        scratch_types=[
            pltpu.VMEM((2, _CHP), jnp.int32),
            pltpu.VMEM((2, _CHP), jnp.int32),
            pltpu.VMEM((_LSZP,), jnp.int32),
            pltpu.VMEM((_LSZP,), jnp.int32),
            pltpu.VMEM((_LSZP,), jnp.int32),
            pltpu.VMEM((_LSZP,), jnp.int32),
            pltpu.VMEM((16,), jnp.int32),
            pltpu.SemaphoreType.DMA((2,)),
            pltpu.SemaphoreType.DMA,
        ],
        compiler_params=pltpu.CompilerParams(needs_layout_passes=False),
    )
    def prep_kernel(src_hbm, dst_hbm, csrc_hbm, coff_hbm, cnt_hbm,
                    dst_v, src_v, csA, csB, coA, coB, cstage, esem, wsem):
        cid = lax.axis_index("c")
        sid = lax.axis_index("s")
        wid = sid * 2 + cid

        def edge_cps(c, q):
            return (pltpu.make_async_copy(
                        dst_hbm.at[pl.ds(c, 1)], dst_v.at[pl.ds(q, 1)],
                        esem.at[q]),
                    pltpu.make_async_copy(
                        src_hbm.at[pl.ds(c, 1)], src_v.at[pl.ds(q, 1)],
                        esem.at[q]))

        def flush_cp(lst, hbm, k, base):
            return pltpu.make_async_copy(
                lst.at[pl.ds(k * 16, 16)],
                hbm.at[pl.ds(base + k * 16, 16)], wsem)

        for p in range(_NTASK // _NTILE):
            task = wid + p * _NTILE
            lo = task * _R
            t2a = task * 2
            t2b = task * 2 + 1

            def clear_body(i, _):
                z = jnp.zeros((16,), jnp.int32)
                tr = jnp.full((16,), _R, jnp.int32)
                csA[pl.ds(i * 16, 16)] = z
                csB[pl.ds(i * 16, 16)] = z
                coA[pl.ds(i * 16, 16)] = tr
                coB[pl.ds(i * 16, 16)] = tr
                return 0
            lax.fori_loop(0, _LSZP // 16, clear_body, 0)

            for cp in edge_cps(0, 0):
                cp.start()

            def chunk_body(c, carry):
                woffa, woffb = carry
                q = c % 2

                @pl.when(c + 1 < _NCHP)
                def _():
                    for cp in edge_cps(c + 1, 1 - q):
                        cp.start()

                for cp in edge_cps(c, q):
                    cp.wait()

                def filt_body(i, cc):
                    ca, cb = cc
                    da = dst_v[q, pl.ds(i * 32, 16)]
                    sa = src_v[q, pl.ds(i * 32, 16)]
                    db = dst_v[q, pl.ds(i * 32 + 16, 16)]
                    sb = src_v[q, pl.ds(i * 32 + 16, 16)]
                    ma = (da >= lo) & (da < lo + _R)
                    mb = (db >= lo) & (db < lo + _R)
                    ia = plsc.cumsum(jnp.where(ma, 1, 0))
                    ib = plsc.cumsum(jnp.where(mb, 1, 0))
                    pa = (ca + ia) - jnp.where(ma, 1, 0)
                    pb = (cb + ib) - jnp.where(mb, 1, 0)
                    plsc.store_scatter(csA, [pa], sa, mask=ma)
                    plsc.store_scatter(coA, [pa], da - lo, mask=ma)
                    plsc.store_scatter(csB, [pb], sb, mask=mb)
                    plsc.store_scatter(coB, [pb], db - lo, mask=mb)
                    return (ca + ia[15], cb + ib[15])

                ca, cb = lax.fori_loop(0, _CHP // 32, filt_body, (0, 0))
                nfa = (ca + 15) // 16
                nfb = (cb + 15) // 16
                basea = pl.multiple_of(t2a * _ECAP + woffa * 16, 16)
                baseb = pl.multiple_of(t2b * _ECAP + woffb * 16, 16)

                def fire_a(k, _):
                    flush_cp(csA, csrc_hbm, k, basea).start()
                    flush_cp(coA, coff_hbm, k, basea).start()
                    return 0
                lax.fori_loop(0, nfa, fire_a, 0)

                def fire_b(k, _):
                    flush_cp(csB, csrc_hbm, k, baseb).start()
                    flush_cp(coB, coff_hbm, k, baseb).start()
                    return 0
                lax.fori_loop(0, nfb, fire_b, 0)

                def wait_a(k, _):
                    flush_cp(csA, csrc_hbm, k, basea).wait()
                    flush_cp(coA, coff_hbm, k, basea).wait()
                    return 0
                lax.fori_loop(0, nfa, wait_a, 0)

                def wait_b(k, _):
                    flush_cp(csB, csrc_hbm, k, baseb).wait()
                    flush_cp(coB, coff_hbm, k, baseb).wait()
                    return 0
                lax.fori_loop(0, nfb, wait_b, 0)

                return (woffa + nfa, woffb + nfb)

            woffa, woffb = lax.fori_loop(0, _NCHP, chunk_body, (0, 0))

            cstage[pl.ds(0, 16)] = jnp.zeros((16,), jnp.int32) + woffa * 16
            pltpu.sync_copy(cstage, cnt_hbm.at[pl.ds(t2a * 16, 16)])
            cstage[pl.ds(0, 16)] = jnp.zeros((16,), jnp.int32) + woffb * 16
            pltpu.sync_copy(cstage, cnt_hbm.at[pl.ds(t2b * 16, 16)])

    return prep_kernel(src.reshape(_NCHP, _CHP), dst.reshape(_NCHP, _CHP))


def _segment_max_sc(norm_h, csrc, coff, cnts):
    mesh = plsc.VectorSubcoreMesh(core_axis_name="c", subcore_axis_name="s")

    @functools.partial(
        pl.kernel,
        mesh=mesh,
        out_type=jax.ShapeDtypeStruct((_NPAD, _D // 2), jnp.int32),
        scratch_types=[
            pltpu.VMEM((2 * _BK,), jnp.int32),
            pltpu.VMEM((2 * _BK,), jnp.int32),
            pltpu.VMEM((4 * 16, _D // 2), jnp.int32),
            pltpu.VMEM((_R + 1, _D // 2), jnp.int32),
            pltpu.VMEM((16,), jnp.int32),
            pltpu.SemaphoreType.DMA((2,)),
            pltpu.SemaphoreType.DMA((4,)),
        ],
        compiler_params=pltpu.CompilerParams(needs_layout_passes=False),
    )
    def seg_kernel(norm_hbm, csrc_hbm, coff_hbm, cnt_hbm, out_hbm,
                   ibuf, obuf, stage_v, acc_v, cstage, isem, gsem):
        cid = lax.axis_index("c")
        sid = lax.axis_index("s")
        wid = sid * 2 + cid

        def blk_cps(t2, ib, qb):
            return (pltpu.make_async_copy(
                        csrc_hbm.at[pl.ds(t2 * _ECAP + ib * _BK, _BK)],
                        ibuf.at[pl.ds(qb * _BK, _BK)], isem.at[qb]),
                    pltpu.make_async_copy(
                        coff_hbm.at[pl.ds(t2 * _ECAP + ib * _BK, _BK)],
                        obuf.at[pl.ds(qb * _BK, _BK)], isem.at[qb]))

        def gather_cp(qb, b, sl):
            return pltpu.make_async_copy(
                norm_hbm.at[ibuf.at[pl.ds(qb * _BK + b * 16, 16)]],
                stage_v.at[pl.ds(sl * 16, 16)], gsem.at[sl])

        for p in range(_NTASK // _NTILE):
            task = wid + p * _NTILE
            lo = task * _R

            neg = plsc.bitcast(jnp.full((32,), _NEG, jnp.bfloat16),
                               jnp.int32)

            def init_body(r, _):
                for j in range(_D // 32):
                    acc_v[r, pl.ds(j * 16, 16)] = neg
                return 0
            lax.fori_loop(0, _R + 1, init_body, 0)

            for l in range(2):
                t2 = task * 2 + l
                pltpu.sync_copy(cnt_hbm.at[pl.ds(t2 * 16, 16)], cstage)
                cv = cstage[pl.ds(0, 16)]
                cnt = cv[0]
                nbk = (cnt + (_BK - 1)) // _BK

                @pl.when(nbk > 0)
                def _():
                    for cp in blk_cps(t2, 0, 0):
                        cp.start()

                def blk_body(ib, _):
                    qb = ib % 2

                    @pl.when(ib + 1 < nbk)
                    def _():
                        for cp in blk_cps(t2, ib + 1, 1 - qb):
                            cp.start()

                    for cp in blk_cps(t2, ib, qb):
                        cp.wait()

                    nb = jnp.minimum(_BK // 16, (cnt - ib * _BK) // 16)

                    def prime(k, _):
                        gather_cp(qb, k, k).start()
                        return 0
                    lax.fori_loop(0, jnp.minimum(nb, 4), prime, 0)

                    def gbody(b, _):
                        sl = b % 4
                        gather_cp(qb, b, sl).wait()
                        ov = obuf[pl.ds(qb * _BK + b * 16, 16)]
                        for e in range(16):
                            off = ov[e]
                            row = sl * 16 + e
                            rv = [plsc.bitcast(
                                      stage_v[row, pl.ds(j * 16, 16)],
                                      jnp.bfloat16)
                                  for j in range(_D // 32)]
                            av = [plsc.bitcast(
                                      acc_v[off, pl.ds(j * 16, 16)],
                                      jnp.bfloat16)
                                  for j in range(_D // 32)]
                            for j in range(_D // 32):
                                acc_v[off, pl.ds(j * 16, 16)] = plsc.bitcast(
                                    jnp.maximum(av[j], rv[j]), jnp.int32)

                        @pl.when(b + 4 < nb)
                        def _():
                            gather_cp(qb, b + 4, sl).start()
                        return 0

                    lax.fori_loop(0, nb, gbody, 0)
                    return 0

                lax.fori_loop(0, nbk, blk_body, 0)

            thr = jnp.full((32,), -1e37, jnp.bfloat16)
            zero = jnp.zeros((32,), jnp.bfloat16)

            def fin_body(r, _):
                for j in range(_D // 32):
                    a = plsc.bitcast(acc_v[r, pl.ds(j * 16, 16)],
                                     jnp.bfloat16)
                    acc_v[r, pl.ds(j * 16, 16)] = plsc.bitcast(
                        jnp.where(a > thr, a, zero), jnp.int32)
                return 0
            lax.fori_loop(0, _R, fin_body, 0)

            pltpu.sync_copy(acc_v.at[pl.ds(0, _R)],
                            out_hbm.at[pl.ds(lo, _R)])

    bits = jax.lax.bitcast_convert_type(
        norm_h.reshape(norm_h.shape[0], _D // 2, 2), jnp.int32)
    out_bits = seg_kernel(bits, csrc, coff, cnts)
    return jax.lax.bitcast_convert_type(
        out_bits, jnp.bfloat16).reshape(_NPAD, _D)


def kernel(x, edge_index, Wagg0, Wagg1, L0, L1):
    src = edge_index[0].astype(jnp.int32)
    dst = edge_index[1].astype(jnp.int32)
    d_in = x.shape[1]

    csrc, coff, cnts = _prep_sc(src, dst)

    norm0 = _matmul(x, Wagg0, jnp.bfloat16)
    hn0 = _segment_max_sc(norm0, csrc, coff, cnts)[:_N]
    h = _concat_matmul(x, hn0, L0[:d_in], L0[d_in:], relu=True)

    norm1 = _matmul(h, Wagg1, jnp.bfloat16)
    hn1 = _segment_max_sc(norm1, csrc, coff, cnts)[:_N]
    d_hid = h.shape[1]
    out = _concat_matmul(h, hn1, L1[:d_hid], L1[d_hid:], relu=False)
    return out


# speedup vs baseline: 1.0199x; 1.0199x over previous
import functools

import jax
import jax.numpy as jnp
from jax import lax
from jax.experimental import pallas as pl
from jax.experimental.pallas import tpu as pltpu
from jax.experimental.pallas import tpu_sc as plsc

_N = 10000
_E = 320000
_D = 512

_NTILE = 32
_NTASK = 64
_R = 160
_NPAD = _NTASK * _R
_CHP = 12800
_NCHP = _E // _CHP
_LSZP = _CHP // 2 + 16
_T2 = _NTASK * 2
_BK = 1024
_ECAP = 160768
_NEG = -3.0e38


def _matmul(a, w, out_dtype=jnp.float32):
    m, k = a.shape
    do = w.shape[1]
    bm = 400

    def body(a_ref, w_ref, o_ref):
        o_ref[...] = jnp.dot(a_ref[...], w_ref[...],
                             preferred_element_type=jnp.float32,
                             precision=jax.lax.Precision.DEFAULT
                             ).astype(out_dtype)

    return pl.pallas_call(
        body,
        grid=(m // bm,),
        in_specs=[
            pl.BlockSpec((bm, k), lambda i: (i, 0)),
            pl.BlockSpec((k, do), lambda i: (0, 0)),
        ],
        out_specs=pl.BlockSpec((bm, do), lambda i: (i, 0)),
        out_shape=jax.ShapeDtypeStruct((m, do), out_dtype),
    )(a, w)


def _concat_matmul(a, b, wt, wb, relu):
    m, ka = a.shape
    kb = b.shape[1]
    do = wt.shape[1]
    bm = 400

    def body(a_ref, b_ref, wt_ref, wb_ref, o_ref):
        acc = jnp.dot(a_ref[...], wt_ref[...],
                      preferred_element_type=jnp.float32,
                      precision=jax.lax.Precision.DEFAULT)
        acc = acc + jnp.dot(b_ref[...].astype(jnp.float32), wb_ref[...],
                            preferred_element_type=jnp.float32,
                            precision=jax.lax.Precision.DEFAULT)
        if relu:
            acc = jnp.maximum(acc, 0.0)
        o_ref[...] = acc

    return pl.pallas_call(
        body,
        grid=(m // bm,),
        in_specs=[
            pl.BlockSpec((bm, ka), lambda i: (i, 0)),
            pl.BlockSpec((bm, kb), lambda i: (i, 0)),
            pl.BlockSpec((ka, do), lambda i: (0, 0)),
            pl.BlockSpec((kb, do), lambda i: (0, 0)),
        ],
        out_specs=pl.BlockSpec((bm, do), lambda i: (i, 0)),
        out_shape=jax.ShapeDtypeStruct((m, do), jnp.float32),
    )(a, b, wt, wb)


def _prep_sc(src, dst):
    mesh = plsc.VectorSubcoreMesh(core_axis_name="c", subcore_axis_name="s")

    @functools.partial(
        pl.kernel,
        mesh=mesh,
        out_type=(
            jax.ShapeDtypeStruct((_T2 * _ECAP,), jnp.int32),
            jax.ShapeDtypeStruct((_T2 * _ECAP,), jnp.int32),
            jax.ShapeDtypeStruct((_T2 * 16,), jnp.int32),
        ),
        scratch_types=[
            pltpu.VMEM((2, _CHP), jnp.int32),
            pltpu.VMEM((2, _CHP), jnp.int32),
            pltpu.VMEM((_LSZP,), jnp.int32),
            pltpu.VMEM((_LSZP,), jnp.int32),
            pltpu.VMEM((_LSZP,), jnp.int32),
            pltpu.VMEM((_LSZP,), jnp.int32),
            pltpu.VMEM((16,), jnp.int32),
            pltpu.SemaphoreType.DMA((2,)),
            pltpu.SemaphoreType.DMA,
        ],
        compiler_params=pltpu.CompilerParams(needs_layout_passes=False),
    )
    def prep_kernel(src_hbm, dst_hbm, csrc_hbm, coff_hbm, cnt_hbm,
                    dst_v, src_v, csA, csB, coA, coB, cstage, esem, wsem):
        cid = lax.axis_index("c")
        sid = lax.axis_index("s")
        wid = sid * 2 + cid

        def edge_cps(c, q):
            return (pltpu.make_async_copy(
                        dst_hbm.at[pl.ds(c, 1)], dst_v.at[pl.ds(q, 1)],
                        esem.at[q]),
                    pltpu.make_async_copy(
                        src_hbm.at[pl.ds(c, 1)], src_v.at[pl.ds(q, 1)],
                        esem.at[q]))

        def flush_cp(lst, hbm, k, base):
            return pltpu.make_async_copy(
                lst.at[pl.ds(k * 16, 16)],
                hbm.at[pl.ds(base + k * 16, 16)], wsem)

        for p in range(_NTASK // _NTILE):
            task = wid + p * _NTILE
            lo = task * _R
            t2a = task * 2
            t2b = task * 2 + 1

            def clear_body(i, _):
                z = jnp.zeros((16,), jnp.int32)
                tr = jnp.full((16,), _R, jnp.int32)
                csA[pl.ds(i * 16, 16)] = z
                csB[pl.ds(i * 16, 16)] = z
                coA[pl.ds(i * 16, 16)] = tr
                coB[pl.ds(i * 16, 16)] = tr
                return 0
            lax.fori_loop(0, _LSZP // 16, clear_body, 0)

            for cp in edge_cps(0, 0):
                cp.start()

            def chunk_body(c, carry):
                woffa, woffb = carry
                q = c % 2

                @pl.when(c + 1 < _NCHP)
                def _():
                    for cp in edge_cps(c + 1, 1 - q):
                        cp.start()

                for cp in edge_cps(c, q):
                    cp.wait()

                def filt_body(i, cc):
                    ca, cb = cc
                    da = dst_v[q, pl.ds(i * 32, 16)]
                    sa = src_v[q, pl.ds(i * 32, 16)]
                    db = dst_v[q, pl.ds(i * 32 + 16, 16)]
                    sb = src_v[q, pl.ds(i * 32 + 16, 16)]
                    ma = (da >= lo) & (da < lo + _R)
                    mb = (db >= lo) & (db < lo + _R)
                    ia = plsc.cumsum(jnp.where(ma, 1, 0))
                    ib = plsc.cumsum(jnp.where(mb, 1, 0))
                    pa = (ca + ia) - jnp.where(ma, 1, 0)
                    pb = (cb + ib) - jnp.where(mb, 1, 0)
                    plsc.store_scatter(csA, [pa], sa, mask=ma)
                    plsc.store_scatter(coA, [pa], da - lo, mask=ma)
                    plsc.store_scatter(csB, [pb], sb, mask=mb)
                    plsc.store_scatter(coB, [pb], db - lo, mask=mb)
                    return (ca + ia[15], cb + ib[15])

                ca, cb = lax.fori_loop(0, _CHP // 32, filt_body, (0, 0))
                nfa = (ca + 15) // 16
                nfb = (cb + 15) // 16
                basea = pl.multiple_of(t2a * _ECAP + woffa * 16, 16)
                baseb = pl.multiple_of(t2b * _ECAP + woffb * 16, 16)

                def fire_a(k, _):
                    flush_cp(csA, csrc_hbm, k, basea).start()
                    flush_cp(coA, coff_hbm, k, basea).start()
                    return 0
                lax.fori_loop(0, nfa, fire_a, 0)

                def fire_b(k, _):
                    flush_cp(csB, csrc_hbm, k, baseb).start()
                    flush_cp(coB, coff_hbm, k, baseb).start()
                    return 0
                lax.fori_loop(0, nfb, fire_b, 0)

                def wait_a(k, _):
                    flush_cp(csA, csrc_hbm, k, basea).wait()
                    flush_cp(coA, coff_hbm, k, basea).wait()
                    return 0
                lax.fori_loop(0, nfa, wait_a, 0)

                def wait_b(k, _):
                    flush_cp(csB, csrc_hbm, k, baseb).wait()
                    flush_cp(coB, coff_hbm, k, baseb).wait()
                    return 0
                lax.fori_loop(0, nfb, wait_b, 0)

                return (woffa + nfa, woffb + nfb)

            woffa, woffb = lax.fori_loop(0, _NCHP, chunk_body, (0, 0))

            cstage[pl.ds(0, 16)] = jnp.zeros((16,), jnp.int32) + woffa * 16
            pltpu.sync_copy(cstage, cnt_hbm.at[pl.ds(t2a * 16, 16)])
            cstage[pl.ds(0, 16)] = jnp.zeros((16,), jnp.int32) + woffb * 16
            pltpu.sync_copy(cstage, cnt_hbm.at[pl.ds(t2b * 16, 16)])

    return prep_kernel(src.reshape(_NCHP, _CHP), dst.reshape(_NCHP, _CHP))


def _segment_max_sc(norm_h, csrc, coff, cnts):
    mesh = plsc.VectorSubcoreMesh(core_axis_name="c", subcore_axis_name="s")

    @functools.partial(
        pl.kernel,
        mesh=mesh,
        out_type=jax.ShapeDtypeStruct((_NPAD, _D // 2), jnp.int32),
        scratch_types=[
            pltpu.VMEM((2 * _BK,), jnp.int32),
            pltpu.VMEM((2 * _BK,), jnp.int32),
            pltpu.VMEM((4 * 16, _D // 2), jnp.int32),
            pltpu.VMEM((_R + 1, _D // 2), jnp.int32),
            pltpu.VMEM((16,), jnp.int32),
            pltpu.SemaphoreType.DMA((2,)),
            pltpu.SemaphoreType.DMA((4,)),
        ],
        compiler_params=pltpu.CompilerParams(needs_layout_passes=False),
    )
    def seg_kernel(norm_hbm, csrc_hbm, coff_hbm, cnt_hbm, out_hbm,
                   ibuf, obuf, stage_v, acc_v, cstage, isem, gsem):
        cid = lax.axis_index("c")
        sid = lax.axis_index("s")
        wid = sid * 2 + cid

        def blk_cps(t2, ib, qb):
            return (pltpu.make_async_copy(
                        csrc_hbm.at[pl.ds(t2 * _ECAP + ib * _BK, _BK)],
                        ibuf.at[pl.ds(qb * _BK, _BK)], isem.at[qb]),
                    pltpu.make_async_copy(
                        coff_hbm.at[pl.ds(t2 * _ECAP + ib * _BK, _BK)],
                        obuf.at[pl.ds(qb * _BK, _BK)], isem.at[qb]))

        def gather_cp(qb, b, sl):
            return pltpu.make_async_copy(
                norm_hbm.at[ibuf.at[pl.ds(qb * _BK + b * 16, 16)]],
                stage_v.at[pl.ds(sl * 16, 16)], gsem.at[sl])

        for p in range(_NTASK // _NTILE):
            task = wid + p * _NTILE
            lo = task * _R

            neg = plsc.bitcast(jnp.full((32,), _NEG, jnp.bfloat16),
                               jnp.int32)

            def init_body(r, _):
                for j in range(_D // 32):
                    acc_v[r, pl.ds(j * 16, 16)] = neg
                return 0
            lax.fori_loop(0, _R + 1, init_body, 0)

            for l in range(2):
                t2 = task * 2 + l
                pltpu.sync_copy(cnt_hbm.at[pl.ds(t2 * 16, 16)], cstage)
                cv = cstage[pl.ds(0, 16)]
                cnt = cv[0]
                nbk = (cnt + (_BK - 1)) // _BK

                @pl.when(nbk > 0)
                def _():
                    for cp in blk_cps(t2, 0, 0):
                        cp.start()

                def blk_body(ib, _):
                    qb = ib % 2

                    @pl.when(ib + 1 < nbk)
                    def _():
                        for cp in blk_cps(t2, ib + 1, 1 - qb):
                            cp.start()

                    for cp in blk_cps(t2, ib, qb):
                        cp.wait()

                    nb = jnp.minimum(_BK // 16, (cnt - ib * _BK) // 16)

                    def prime(k, _):
                        gather_cp(qb, k, k).start()
                        return 0
                    lax.fori_loop(0, jnp.minimum(nb, 4), prime, 0)

                    def gbody(b, _):
                        sl = b % 4
                        gather_cp(qb, b, sl).wait()
                        ov = obuf[pl.ds(qb * _BK + b * 16, 16)]
                        for e in range(16):
                            off = ov[e]
                            row = sl * 16 + e
                            rv = [plsc.bitcast(
                                      stage_v[row, pl.ds(j * 16, 16)],
                                      jnp.bfloat16)
                                  for j in range(_D // 32)]
                            av = [plsc.bitcast(
                                      acc_v[off, pl.ds(j * 16, 16)],
                                      jnp.bfloat16)
                                  for j in range(_D // 32)]
                            for j in range(_D // 32):
                                acc_v[off, pl.ds(j * 16, 16)] = plsc.bitcast(
                                    jnp.maximum(av[j], rv[j]), jnp.int32)

                        @pl.when(b + 4 < nb)
                        def _():
                            gather_cp(qb, b + 4, sl).start()
                        return 0

                    lax.fori_loop(0, nb, gbody, 0)
                    return 0

                lax.fori_loop(0, nbk, blk_body, 0)

            thr = jnp.full((32,), -1e37, jnp.bfloat16)
            zero = jnp.zeros((32,), jnp.bfloat16)

            def fin_body(r, _):
                for j in range(_D // 32):
                    a = plsc.bitcast(acc_v[r, pl.ds(j * 16, 16)],
                                     jnp.bfloat16)
                    acc_v[r, pl.ds(j * 16, 16)] = plsc.bitcast(
                        jnp.where(a > thr, a, zero), jnp.int32)
                return 0
            lax.fori_loop(0, _R, fin_body, 0)

            pltpu.sync_copy(acc_v.at[pl.ds(0, _R)],
                            out_hbm.at[pl.ds(lo, _R)])

    bits = jax.lax.bitcast_convert_type(
        norm_h.reshape(norm_h.shape[0], _D // 2, 2), jnp.int32)
    out_bits = seg_kernel(bits, csrc, coff, cnts)
    return jax.lax.bitcast_convert_type(
        out_bits, jnp.bfloat16).reshape(_NPAD, _D)


def kernel(x, edge_index, Wagg0, Wagg1, L0, L1):
    src = edge_index[0].astype(jnp.int32)
    dst = edge_index[1].astype(jnp.int32)
    d_in = x.shape[1]

    csrc, coff, cnts = _prep_sc(src, dst)

    norm0 = _matmul(x, Wagg0, jnp.bfloat16)
    hn0 = _segment_max_sc(norm0, csrc, coff, cnts)[:_N]
    h = _concat_matmul(x, hn0, L0[:d_in], L0[d_in:], relu=True)

    norm1 = _matmul(h, Wagg1, jnp.bfloat16)
    hn1 = _segment_max_sc(norm1, csrc, coff, cnts)[:_N]
    d_hid = h.shape[1]
    out = _concat_matmul(h, hn1, L1[:d_hid], L1[d_hid:], relu=False)
    return out


# speedup vs baseline: 1.5095x; 1.4801x over previous
import functools

import jax
import jax.numpy as jnp
from jax import lax
from jax.experimental import pallas as pl
from jax.experimental.pallas import tpu as pltpu
from jax.experimental.pallas import tpu_sc as plsc

_N = 10000
_E = 320000
_D = 512

_NTILE = 32
_NTASK = 64
_R = 160
_NPAD = _NTASK * _R
_CHP = 12800
_NCHP = _E // _CHP
_LSZP = _CHP // 2 + 16
_T2 = _NTASK * 2
_BK = 1024
_ECAP = 160768
_NEG = -3.0e38


def _matmul_packed(a, w):
    m, k = a.shape
    do = w.shape[1]
    hd = do // 2
    bm = 400

    def body(a_ref, w_ref, o_ref):
        acc = jnp.dot(a_ref[...], w_ref[...],
                      preferred_element_type=jnp.float32,
                      precision=jax.lax.Precision.DEFAULT)
        lo = jax.lax.bitcast_convert_type(
            acc[:, :hd].astype(jnp.bfloat16), jnp.uint16).astype(jnp.uint32)
        hi = jax.lax.bitcast_convert_type(
            acc[:, hd:].astype(jnp.bfloat16), jnp.uint16).astype(jnp.uint32)
        o_ref[...] = jax.lax.bitcast_convert_type(
            (hi << 16) | lo, jnp.int32)

    return pl.pallas_call(
        body,
        grid=(m // bm,),
        in_specs=[
            pl.BlockSpec((bm, k), lambda i: (i, 0)),
            pl.BlockSpec((k, do), lambda i: (0, 0)),
        ],
        out_specs=pl.BlockSpec((bm, hd), lambda i: (i, 0)),
        out_shape=jax.ShapeDtypeStruct((m, hd), jnp.int32),
    )(a, w)


def _concat_matmul(a, b, wt, wb_lo, wb_hi, relu):
    m, ka = a.shape
    kb = b.shape[1]
    do = wt.shape[1]
    bm = 400

    def body(a_ref, b_ref, wt_ref, wbl_ref, wbh_ref, o_ref):
        acc = jnp.dot(a_ref[...], wt_ref[...],
                      preferred_element_type=jnp.float32,
                      precision=jax.lax.Precision.DEFAULT)
        w = jax.lax.bitcast_convert_type(b_ref[...], jnp.uint32)
        blo = jax.lax.bitcast_convert_type(
            (w & 0xFFFF).astype(jnp.uint16), jnp.bfloat16).astype(jnp.float32)
        bhi = jax.lax.bitcast_convert_type(
            (w >> 16).astype(jnp.uint16), jnp.bfloat16).astype(jnp.float32)
        acc = acc + jnp.dot(blo, wbl_ref[...],
                            preferred_element_type=jnp.float32,
                            precision=jax.lax.Precision.DEFAULT)
        acc = acc + jnp.dot(bhi, wbh_ref[...],
                            preferred_element_type=jnp.float32,
                            precision=jax.lax.Precision.DEFAULT)
        if relu:
            acc = jnp.maximum(acc, 0.0)
        o_ref[...] = acc

    return pl.pallas_call(
        body,
        grid=(m // bm,),
        in_specs=[
            pl.BlockSpec((bm, ka), lambda i: (i, 0)),
            pl.BlockSpec((bm, kb), lambda i: (i, 0)),
            pl.BlockSpec((ka, do), lambda i: (0, 0)),
            pl.BlockSpec((kb // 1, do), lambda i: (0, 0)),
            pl.BlockSpec((kb // 1, do), lambda i: (0, 0)),
        ],
        out_specs=pl.BlockSpec((bm, do), lambda i: (i, 0)),
        out_shape=jax.ShapeDtypeStruct((m, do), jnp.float32),
    )(a, b, wt, wb_lo, wb_hi)


def _prep_sc(src, dst):
    mesh = plsc.VectorSubcoreMesh(core_axis_name="c", subcore_axis_name="s")

    @functools.partial(
        pl.kernel,
        mesh=mesh,
        out_type=(
            jax.ShapeDtypeStruct((_T2 * _ECAP,), jnp.int32),
            jax.ShapeDtypeStruct((_T2 * _ECAP,), jnp.int32),
            jax.ShapeDtypeStruct((_T2 * 16,), jnp.int32),
        ),
        scratch_types=[
            pltpu.VMEM((2, _CHP), jnp.int32),
            pltpu.VMEM((2, _CHP), jnp.int32),
            pltpu.VMEM((_LSZP,), jnp.int32),
            pltpu.VMEM((_LSZP,), jnp.int32),
            pltpu.VMEM((_LSZP,), jnp.int32),
            pltpu.VMEM((_LSZP,), jnp.int32),
            pltpu.VMEM((16,), jnp.int32),
            pltpu.SemaphoreType.DMA((2,)),
            pltpu.SemaphoreType.DMA,
        ],
        compiler_params=pltpu.CompilerParams(needs_layout_passes=False),
    )
    def prep_kernel(src_hbm, dst_hbm, csrc_hbm, coff_hbm, cnt_hbm,
                    dst_v, src_v, csA, csB, coA, coB, cstage, esem, wsem):
        cid = lax.axis_index("c")
        sid = lax.axis_index("s")
        wid = sid * 2 + cid

        def edge_cps(c, q):
            return (pltpu.make_async_copy(
                        dst_hbm.at[pl.ds(c, 1)], dst_v.at[pl.ds(q, 1)],
                        esem.at[q]),
                    pltpu.make_async_copy(
                        src_hbm.at[pl.ds(c, 1)], src_v.at[pl.ds(q, 1)],
                        esem.at[q]))

        def flush_cp(lst, hbm, k, base):
            return pltpu.make_async_copy(
                lst.at[pl.ds(k * 16, 16)],
                hbm.at[pl.ds(base + k * 16, 16)], wsem)

        for p in range(_NTASK // _NTILE):
            task = wid + p * _NTILE
            lo = task * _R
            t2a = task * 2
            t2b = task * 2 + 1

            def clear_body(i, _):
                z = jnp.zeros((16,), jnp.int32)
                tr = jnp.full((16,), _R, jnp.int32)
                csA[pl.ds(i * 16, 16)] = z
                csB[pl.ds(i * 16, 16)] = z
                coA[pl.ds(i * 16, 16)] = tr
                coB[pl.ds(i * 16, 16)] = tr
                return 0
            lax.fori_loop(0, _LSZP // 16, clear_body, 0)

            for cp in edge_cps(0, 0):
                cp.start()

            def chunk_body(c, carry):
                woffa, woffb = carry
                q = c % 2

                @pl.when(c + 1 < _NCHP)
                def _():
                    for cp in edge_cps(c + 1, 1 - q):
                        cp.start()

                for cp in edge_cps(c, q):
                    cp.wait()

                def filt_body(i, cc):
                    ca, cb = cc
                    da = dst_v[q, pl.ds(i * 32, 16)]
                    sa = src_v[q, pl.ds(i * 32, 16)]
                    db = dst_v[q, pl.ds(i * 32 + 16, 16)]
                    sb = src_v[q, pl.ds(i * 32 + 16, 16)]
                    ma = (da >= lo) & (da < lo + _R)
                    mb = (db >= lo) & (db < lo + _R)
                    ia = plsc.cumsum(jnp.where(ma, 1, 0))
                    ib = plsc.cumsum(jnp.where(mb, 1, 0))
                    pa = (ca + ia) - jnp.where(ma, 1, 0)
                    pb = (cb + ib) - jnp.where(mb, 1, 0)
                    plsc.store_scatter(csA, [pa], sa, mask=ma)
                    plsc.store_scatter(coA, [pa], da - lo, mask=ma)
                    plsc.store_scatter(csB, [pb], sb, mask=mb)
                    plsc.store_scatter(coB, [pb], db - lo, mask=mb)
                    return (ca + ia[15], cb + ib[15])

                ca, cb = lax.fori_loop(0, _CHP // 32, filt_body, (0, 0))
                nfa = (ca + 15) // 16
                nfb = (cb + 15) // 16
                basea = pl.multiple_of(t2a * _ECAP + woffa * 16, 16)
                baseb = pl.multiple_of(t2b * _ECAP + woffb * 16, 16)

                def fire_a(k, _):
                    flush_cp(csA, csrc_hbm, k, basea).start()
                    flush_cp(coA, coff_hbm, k, basea).start()
                    return 0
                lax.fori_loop(0, nfa, fire_a, 0)

                def fire_b(k, _):
                    flush_cp(csB, csrc_hbm, k, baseb).start()
                    flush_cp(coB, coff_hbm, k, baseb).start()
                    return 0
                lax.fori_loop(0, nfb, fire_b, 0)

                def wait_a(k, _):
                    flush_cp(csA, csrc_hbm, k, basea).wait()
                    flush_cp(coA, coff_hbm, k, basea).wait()
                    return 0
                lax.fori_loop(0, nfa, wait_a, 0)

                def wait_b(k, _):
                    flush_cp(csB, csrc_hbm, k, baseb).wait()
                    flush_cp(coB, coff_hbm, k, baseb).wait()
                    return 0
                lax.fori_loop(0, nfb, wait_b, 0)

                return (woffa + nfa, woffb + nfb)

            woffa, woffb = lax.fori_loop(0, _NCHP, chunk_body, (0, 0))

            cstage[pl.ds(0, 16)] = jnp.zeros((16,), jnp.int32) + woffa * 16
            pltpu.sync_copy(cstage, cnt_hbm.at[pl.ds(t2a * 16, 16)])
            cstage[pl.ds(0, 16)] = jnp.zeros((16,), jnp.int32) + woffb * 16
            pltpu.sync_copy(cstage, cnt_hbm.at[pl.ds(t2b * 16, 16)])

    return prep_kernel(src.reshape(_NCHP, _CHP), dst.reshape(_NCHP, _CHP))


def _segment_max_sc(norm_h, csrc, coff, cnts):
    mesh = plsc.VectorSubcoreMesh(core_axis_name="c", subcore_axis_name="s")

    @functools.partial(
        pl.kernel,
        mesh=mesh,
        out_type=jax.ShapeDtypeStruct((_NPAD, _D // 2), jnp.int32),
        scratch_types=[
            pltpu.VMEM((2 * _BK,), jnp.int32),
            pltpu.VMEM((2 * _BK,), jnp.int32),
            pltpu.VMEM((4 * 16, _D // 2), jnp.int32),
            pltpu.VMEM((_R + 1, _D // 2), jnp.int32),
            pltpu.VMEM((16,), jnp.int32),
            pltpu.SemaphoreType.DMA((2,)),
            pltpu.SemaphoreType.DMA((4,)),
        ],
        compiler_params=pltpu.CompilerParams(needs_layout_passes=False),
    )
    def seg_kernel(norm_hbm, csrc_hbm, coff_hbm, cnt_hbm, out_hbm,
                   ibuf, obuf, stage_v, acc_v, cstage, isem, gsem):
        cid = lax.axis_index("c")
        sid = lax.axis_index("s")
        wid = sid * 2 + cid

        def blk_cps(t2, ib, qb):
            return (pltpu.make_async_copy(
                        csrc_hbm.at[pl.ds(t2 * _ECAP + ib * _BK, _BK)],
                        ibuf.at[pl.ds(qb * _BK, _BK)], isem.at[qb]),
                    pltpu.make_async_copy(
                        coff_hbm.at[pl.ds(t2 * _ECAP + ib * _BK, _BK)],
                        obuf.at[pl.ds(qb * _BK, _BK)], isem.at[qb]))

        def gather_cp(qb, b, sl):
            return pltpu.make_async_copy(
                norm_hbm.at[ibuf.at[pl.ds(qb * _BK + b * 16, 16)]],
                stage_v.at[pl.ds(sl * 16, 16)], gsem.at[sl])

        for p in range(_NTASK // _NTILE):
            task = wid + p * _NTILE
            lo = task * _R

            neg = plsc.bitcast(jnp.full((32,), _NEG, jnp.bfloat16),
                               jnp.int32)

            def init_body(r, _):
                for j in range(_D // 32):
                    acc_v[r, pl.ds(j * 16, 16)] = neg
                return 0
            lax.fori_loop(0, _R + 1, init_body, 0)

            for l in range(2):
                t2 = task * 2 + l
                pltpu.sync_copy(cnt_hbm.at[pl.ds(t2 * 16, 16)], cstage)
                cv = cstage[pl.ds(0, 16)]
                cnt = cv[0]
                nbk = (cnt + (_BK - 1)) // _BK

                @pl.when(nbk > 0)
                def _():
                    for cp in blk_cps(t2, 0, 0):
                        cp.start()

                def blk_body(ib, _):
                    qb = ib % 2

                    @pl.when(ib + 1 < nbk)
                    def _():
                        for cp in blk_cps(t2, ib + 1, 1 - qb):
                            cp.start()

                    for cp in blk_cps(t2, ib, qb):
                        cp.wait()

                    nb = jnp.minimum(_BK // 16, (cnt - ib * _BK) // 16)

                    def prime(k, _):
                        gather_cp(qb, k, k).start()
                        return 0
                    lax.fori_loop(0, jnp.minimum(nb, 4), prime, 0)

                    def gbody(b, _):
                        sl = b % 4
                        gather_cp(qb, b, sl).wait()
                        ov = obuf[pl.ds(qb * _BK + b * 16, 16)]
                        for e in range(16):
                            off = ov[e]
                            row = sl * 16 + e
                            rv = [plsc.bitcast(
                                      stage_v[row, pl.ds(j * 16, 16)],
                                      jnp.bfloat16)
                                  for j in range(_D // 32)]
                            av = [plsc.bitcast(
                                      acc_v[off, pl.ds(j * 16, 16)],
                                      jnp.bfloat16)
                                  for j in range(_D // 32)]
                            for j in range(_D // 32):
                                acc_v[off, pl.ds(j * 16, 16)] = plsc.bitcast(
                                    jnp.maximum(av[j], rv[j]), jnp.int32)

                        @pl.when(b + 4 < nb)
                        def _():
                            gather_cp(qb, b + 4, sl).start()
                        return 0

                    lax.fori_loop(0, nb, gbody, 0)
                    return 0

                lax.fori_loop(0, nbk, blk_body, 0)

            thr = jnp.full((32,), -1e37, jnp.bfloat16)
            zero = jnp.zeros((32,), jnp.bfloat16)

            def fin_body(r, _):
                for j in range(_D // 32):
                    a = plsc.bitcast(acc_v[r, pl.ds(j * 16, 16)],
                                     jnp.bfloat16)
                    acc_v[r, pl.ds(j * 16, 16)] = plsc.bitcast(
                        jnp.where(a > thr, a, zero), jnp.int32)
                return 0
            lax.fori_loop(0, _R, fin_body, 0)

            pltpu.sync_copy(acc_v.at[pl.ds(0, _R)],
                            out_hbm.at[pl.ds(lo, _R)])

    return seg_kernel(norm_h, csrc, coff, cnts)


def kernel(x, edge_index, Wagg0, Wagg1, L0, L1):
    src = edge_index[0].astype(jnp.int32)
    dst = edge_index[1].astype(jnp.int32)
    d_in = x.shape[1]

    csrc, coff, cnts = _prep_sc(src, dst)

    hd = _D // 2

    norm0 = _matmul_packed(x, Wagg0)
    hn0 = _segment_max_sc(norm0, csrc, coff, cnts)[:_N]
    wb0 = L0[d_in:]
    h = _concat_matmul(x, hn0, L0[:d_in], wb0[:hd], wb0[hd:],
                       relu=True)

    norm1 = _matmul_packed(h, Wagg1)
    hn1 = _segment_max_sc(norm1, csrc, coff, cnts)[:_N]
    d_hid = h.shape[1]
    wb1 = L1[d_hid:]
    out = _concat_matmul(h, hn1, L1[:d_hid], wb1[:hd], wb1[hd:],
                         relu=False)
    return out
